# Initial kernel scaffold; baseline (speedup 1.0000x reference)
import jax
import jax.numpy as jnp
from jax.experimental import pallas as pl


def kernel(x, centroids):
    raise NotImplementedError("write your pallas kernel here")


# speedup vs baseline: 1.1421x; 1.1421x over previous
import functools

import jax
import jax.numpy as jnp
from jax import lax
from jax.experimental import pallas as pl
from jax.experimental.pallas import tpu as pltpu
from jax.experimental.pallas import tpu_sc as plsc

_N = 16384
_D = 32
_K = 8192

_BN = 256


def _argmin_body(ct_ref, x_ref, xsq_ref, csq_ref, idx_ref):
    ct = ct_ref[...]
    x = x_ref[...]
    mmT = lax.dot_general(ct, x, (((1,), (1,)), ((), ())),
                          preferred_element_type=jnp.float32)
    dT = (csq_ref[...] + xsq_ref[...]) - 2.0 * mmT
    minv = jnp.min(dT, axis=0, keepdims=True)
    kio = lax.broadcasted_iota(jnp.int32, dT.shape, 0)
    idx = jnp.min(jnp.where(dT == minv, kio, _K), axis=0)
    idx_ref[...] = idx[None, None, :]


def _tc_argmin(x, table, csq):
    xsq = jnp.sum(x * x, axis=1)[None, :]
    idx = pl.pallas_call(
        _argmin_body,
        grid=(_N // _BN,),
        in_specs=[
            pl.BlockSpec((_K, _D), lambda i: (0, 0)),
            pl.BlockSpec((_BN, _D), lambda i: (i, 0)),
            pl.BlockSpec((1, _BN), lambda i: (0, i)),
            pl.BlockSpec((_K, 1), lambda i: (0, 0)),
        ],
        out_specs=pl.BlockSpec((1, 1, _BN), lambda i: (i, 0, 0)),
        out_shape=jax.ShapeDtypeStruct((_N // _BN, 1, _BN), jnp.int32),
    )(table, x, xsq, csq)
    return idx.reshape(_N)


_NC, _NS = 2, 16
_NW = _NC * _NS
_BPW = _N // _NW
_CB = 128
_CHUNKS = _BPW // _CB


@functools.cache
def _make_sc_gather():
    mesh = plsc.VectorSubcoreMesh(core_axis_name="c", subcore_axis_name="s")

    @functools.partial(
        pl.kernel,
        mesh=mesh,
        compiler_params=pltpu.CompilerParams(use_tc_tiling_on_sc=False),
        out_type=jax.ShapeDtypeStruct((_N, _D), jnp.float32),
        scratch_types=[
            pltpu.VMEM((_CHUNKS, _CB), jnp.int32),
            pltpu.VMEM((_CHUNKS, _CB, _D), jnp.float32),
            pltpu.SemaphoreType.DMA,
        ],
    )
    def _sc_gather(table_hbm, idx_hbm, out_hbm, idx_v, rows_v, sem):
        wid = lax.axis_index("s") * _NC + lax.axis_index("c")
        base = wid * _BPW
        for j in range(_CHUNKS):
            pltpu.sync_copy(idx_hbm.at[pl.ds(base + j * _CB, _CB)], idx_v.at[j])
        copies = [
            pltpu.async_copy(table_hbm.at[idx_v.at[j]], rows_v.at[j], sem)
            for j in range(_CHUNKS)
        ]
        for j in range(_CHUNKS):
            copies[j].wait()
        for j in range(_CHUNKS):
            pltpu.sync_copy(rows_v.at[j], out_hbm.at[pl.ds(base + j * _CB, _CB)])

    return _sc_gather


def kernel(x, centroids):
    table = centroids.T
    csq = jnp.sum(centroids * centroids, axis=0)[:, None]
    idx = _tc_argmin(x, table, csq)
    return _make_sc_gather()(table, idx)


# speedup vs baseline: 1.2750x; 1.1163x over previous
import functools

import jax
import jax.numpy as jnp
from jax import lax
from jax.experimental import pallas as pl
from jax.experimental.pallas import tpu as pltpu
from jax.experimental.pallas import tpu_sc as plsc

_N = 16384
_D = 32
_K = 8192

_BN = 256


def _argmin_body(x_ref, xsq_ref, c_ref, idx_ref):
    x = x_ref[...]
    c = c_ref[...]
    xsq = xsq_ref[...]
    csq = jnp.sum(c * c, axis=0, keepdims=True)
    mm = lax.dot_general(x, c, (((1,), (0,)), ((), ())),
                         preferred_element_type=jnp.float32)
    d = (csq + xsq) - 2.0 * mm
    minv = jnp.min(d, axis=1, keepdims=True)
    kio = lax.broadcasted_iota(jnp.int32, d.shape, 1)
    idx = jnp.min(jnp.where(d == minv, kio, _K), axis=1)
    idx_ref[...] = idx[:, None]


def _tc_argmin(x, centroids):
    xsq = jnp.sum(x * x, axis=1, keepdims=True)
    idx = pl.pallas_call(
        _argmin_body,
        grid=(_N // _BN,),
        in_specs=[
            pl.BlockSpec((_BN, _D), lambda i: (i, 0)),
            pl.BlockSpec((_BN, 1), lambda i: (i, 0)),
            pl.BlockSpec((_D, _K), lambda i: (0, 0)),
        ],
        out_specs=pl.BlockSpec((_BN, 1), lambda i: (i, 0)),
        out_shape=jax.ShapeDtypeStruct((_N, 1), jnp.int32),
    )(x, xsq, centroids)
    return idx.reshape(_N)


_NC, _NS = 2, 16
_NW = _NC * _NS
_BPW = _N // _NW
_CB = 128
_CHUNKS = _BPW // _CB


@functools.cache
def _make_sc_gather():
    mesh = plsc.VectorSubcoreMesh(core_axis_name="c", subcore_axis_name="s")

    @functools.partial(
        pl.kernel,
        mesh=mesh,
        compiler_params=pltpu.CompilerParams(use_tc_tiling_on_sc=False),
        out_type=jax.ShapeDtypeStruct((_N, _D), jnp.float32),
        scratch_types=[
            pltpu.VMEM((_CHUNKS, _CB), jnp.int32),
            pltpu.VMEM((_CHUNKS, _CB, _D), jnp.float32),
            pltpu.SemaphoreType.DMA,
        ],
    )
    def _sc_gather(table_hbm, idx_hbm, out_hbm, idx_v, rows_v, sem):
        wid = lax.axis_index("s") * _NC + lax.axis_index("c")
        base = wid * _BPW
        for j in range(_CHUNKS):
            pltpu.sync_copy(idx_hbm.at[pl.ds(base + j * _CB, _CB)], idx_v.at[j])
        copies = [
            pltpu.async_copy(table_hbm.at[idx_v.at[j]], rows_v.at[j], sem)
            for j in range(_CHUNKS)
        ]
        for j in range(_CHUNKS):
            copies[j].wait()
        for j in range(_CHUNKS):
            pltpu.sync_copy(rows_v.at[j], out_hbm.at[pl.ds(base + j * _CB, _CB)])

    return _sc_gather


def kernel(x, centroids):
    table = centroids.T
    idx = _tc_argmin(x, centroids)
    return _make_sc_gather()(table, idx)


# speedup vs baseline: 1.3474x; 1.0568x over previous
import functools

import jax
import jax.numpy as jnp
from jax import lax
from jax.experimental import pallas as pl
from jax.experimental.pallas import tpu as pltpu
from jax.experimental.pallas import tpu_sc as plsc

_N = 16384
_D = 32
_K = 8192

_BN = 512


def _argmin_body(x_ref, xsq_ref, c_ref, idx_ref):
    x = x_ref[...]
    c = c_ref[...]
    xsq = xsq_ref[...]
    csq = jnp.sum(c * c, axis=0, keepdims=True)
    mm = lax.dot_general(x, c, (((1,), (0,)), ((), ())),
                         preferred_element_type=jnp.float32)
    d = (csq + xsq) - 2.0 * mm
    minv = jnp.min(d, axis=1, keepdims=True)
    kio = lax.broadcasted_iota(jnp.int32, d.shape, 1)
    idx = jnp.min(jnp.where(d == minv, kio, _K), axis=1)
    idx_ref[...] = idx[:, None]


def _tc_argmin(x, centroids):
    xsq = jnp.sum(x * x, axis=1, keepdims=True)
    idx = pl.pallas_call(
        _argmin_body,
        grid=(_N // _BN,),
        in_specs=[
            pl.BlockSpec((_BN, _D), lambda i: (i, 0)),
            pl.BlockSpec((_BN, 1), lambda i: (i, 0)),
            pl.BlockSpec((_D, _K), lambda i: (0, 0)),
        ],
        out_specs=pl.BlockSpec((_BN, 1), lambda i: (i, 0)),
        out_shape=jax.ShapeDtypeStruct((_N, 1), jnp.int32),
    )(x, xsq, centroids)
    return idx.reshape(_N)


_NC, _NS = 2, 16
_NW = _NC * _NS
_BPW = _N // _NW
_CB = 128
_CHUNKS = _BPW // _CB


@functools.cache
def _make_sc_gather():
    mesh = plsc.VectorSubcoreMesh(core_axis_name="c", subcore_axis_name="s")

    @functools.partial(
        pl.kernel,
        mesh=mesh,
        compiler_params=pltpu.CompilerParams(use_tc_tiling_on_sc=False),
        out_type=jax.ShapeDtypeStruct((_N, _D), jnp.float32),
        scratch_types=[
            pltpu.VMEM((_CHUNKS, _CB), jnp.int32),
            pltpu.VMEM((_CHUNKS, _CB, _D), jnp.float32),
            pltpu.SemaphoreType.DMA,
        ],
    )
    def _sc_gather(table_hbm, idx_hbm, out_hbm, idx_v, rows_v, sem):
        wid = lax.axis_index("s") * _NC + lax.axis_index("c")
        base = wid * _BPW
        for j in range(_CHUNKS):
            pltpu.sync_copy(idx_hbm.at[pl.ds(base + j * _CB, _CB)], idx_v.at[j])
        copies = [
            pltpu.async_copy(table_hbm.at[idx_v.at[j]], rows_v.at[j], sem)
            for j in range(_CHUNKS)
        ]
        for j in range(_CHUNKS):
            copies[j].wait()
        for j in range(_CHUNKS):
            pltpu.sync_copy(rows_v.at[j], out_hbm.at[pl.ds(base + j * _CB, _CB)])

    return _sc_gather


def kernel(x, centroids):
    table = centroids.T
    idx = _tc_argmin(x, centroids)
    return _make_sc_gather()(table, idx)
